# speedup vs baseline: 11.1835x; 11.1835x over previous
import functools

import jax
import jax.numpy as jnp
from jax.experimental import pallas as pl
from jax.experimental.pallas import tpu as pltpu

DIM = 2048
BITS = 64
HEXPERTS = 4


def _fused_body(x_ref, wt_ref, b1_ref, emb_ref, wa_ref, ba_ref, out_ref):
    x = x_ref[...]
    logits = jnp.dot(x, wt_ref[...],
                     preferred_element_type=jnp.float32) + b1_ref[...]
    m = jnp.max(logits, axis=-1, keepdims=True)
    e = jnp.exp(logits - m)
    h = e / jnp.sum(e, axis=-1, keepdims=True)

    iota = jax.lax.broadcasted_iota(jnp.int32, h.shape, 1)
    c = jnp.zeros_like(h)
    hw = h
    for rank in range(HEXPERTS):
        mx = jnp.max(hw, axis=-1, keepdims=True)
        eq = hw == mx
        sel = jnp.min(jnp.where(eq, iota, BITS), axis=-1, keepdims=True)
        onehot = iota == sel
        c = c + jnp.where(onehot, wa_ref[0, rank] * mx, 0.0)
        hw = jnp.where(onehot, -1.0, hw)

    comb = jnp.dot(c, emb_ref[...],
                   preferred_element_type=jnp.float32) + ba_ref[0, 0]
    out_ref[...] = x * comb


@functools.partial(jax.jit, static_argnames=("block_t",))
def _run(x2d, wt, b1, emb, wa, ba, block_t=512):
    n_tokens = x2d.shape[0]
    grid = (n_tokens // block_t,)
    return pl.pallas_call(
        _fused_body,
        grid=grid,
        in_specs=[
            pl.BlockSpec((block_t, DIM), lambda i: (i, 0)),
            pl.BlockSpec((DIM, BITS), lambda i: (0, 0)),
            pl.BlockSpec((1, BITS), lambda i: (0, 0)),
            pl.BlockSpec((BITS, DIM), lambda i: (0, 0)),
            pl.BlockSpec((1, HEXPERTS), lambda i: (0, 0)),
            pl.BlockSpec((1, 1), lambda i: (0, 0)),
        ],
        out_specs=pl.BlockSpec((block_t, DIM), lambda i: (i, 0)),
        out_shape=jax.ShapeDtypeStruct((n_tokens, DIM), jnp.float32),
        compiler_params=pltpu.CompilerParams(
            dimension_semantics=("arbitrary",),
        ),
    )(x2d, wt, b1, emb, wa, ba)


def kernel(x, W_a1, b_a1, emb, W_aggr, b_aggr):
    B, S, _ = x.shape
    x2d = x.reshape(B * S, DIM)
    out = _run(x2d, W_a1.T, b_a1.reshape(1, BITS), emb,
               W_aggr.reshape(1, HEXPERTS), b_aggr.reshape(1, 1))
    return out.reshape(B, S, DIM)
